# Initial kernel scaffold; baseline (speedup 1.0000x reference)
import jax
import jax.numpy as jnp
from jax.experimental import pallas as pl


def kernel(x, W, b):
    raise NotImplementedError("write your pallas kernel here")


# speedup vs baseline: 2.5149x; 2.5149x over previous
import functools

import jax
import jax.numpy as jnp
from jax import lax
from jax.experimental import pallas as pl
from jax.experimental.pallas import tpu as pltpu

_DIM = 4096
_NE = 64
_TB = 512


def _router_body(x_ref, w_ref, b_ref, probs_ref, wts_ref, idx_ref):
    x = x_ref[...]
    w = w_ref[...]
    logits = lax.dot_general(x, w, (((1,), (1,)), ((), ())))
    logits = logits + b_ref[...]
    m = jnp.max(logits, axis=1, keepdims=True)
    e = jnp.exp(logits - m)
    s = jnp.sum(e, axis=1, keepdims=True)
    probs = e / s
    probs_ref[...] = probs
    pm = jnp.max(probs, axis=1, keepdims=True)
    wts_ref[...] = pm
    ii = lax.broadcasted_iota(jnp.int32, probs.shape, 1)
    cand = jnp.where(probs == pm, ii, _NE)
    idx_ref[...] = jnp.min(cand, axis=1, keepdims=True)


def kernel(x, W, b):
    ntok = x.shape[0]
    grid = (ntok // _TB,)
    probs, wts, idx = pl.pallas_call(
        _router_body,
        grid=grid,
        in_specs=[
            pl.BlockSpec((_TB, _DIM), lambda i: (i, 0)),
            pl.BlockSpec((_NE, _DIM), lambda i: (0, 0)),
            pl.BlockSpec((1, _NE), lambda i: (0, 0)),
        ],
        out_specs=[
            pl.BlockSpec((_TB, _NE), lambda i: (i, 0)),
            pl.BlockSpec((_TB, 1), lambda i: (i, 0)),
            pl.BlockSpec((_TB, 1), lambda i: (i, 0)),
        ],
        out_shape=[
            jax.ShapeDtypeStruct((ntok, _NE), jnp.float32),
            jax.ShapeDtypeStruct((ntok, 1), jnp.float32),
            jax.ShapeDtypeStruct((ntok, 1), jnp.int32),
        ],
    )(x, W, b.reshape(1, _NE))
    return (wts, idx, probs)
